# baseline (speedup 1.0000x reference)
import functools

import jax
import jax.numpy as jnp
from jax import lax
from jax.experimental import pallas as pl
from jax.experimental.pallas import tpu as pltpu
from jax.experimental.pallas import tpu_sc as plsc

NC, NS, L = 2, 16, 16
NW = NC * NS


def _ceil_to(x, m):
    return (x + m - 1) // m * m


@functools.lru_cache(maxsize=None)
def _make_kernel(n_obs, n_cam, n_pts):
    B = 3136
    nsteps = -(-n_obs // (NW * B))
    C = nsteps * B
    S = _ceil_to(-(-(n_obs - C) // (NW - 1)), 16)
    last = n_obs - C

    mesh = plsc.VectorSubcoreMesh(
        core_axis_name="c", subcore_axis_name="s",
        num_cores=NC, num_subcores=NS)

    @functools.partial(
        pl.kernel,
        out_type=(jax.ShapeDtypeStruct((n_obs,), jnp.float32),
                  jax.ShapeDtypeStruct((n_obs,), jnp.float32)),
        mesh=mesh,
        compiler_params=pltpu.CompilerParams(
            needs_layout_passes=False, use_tc_tiling_on_sc=False,
            disable_bounds_checks=True),
        scratch_types=[
            pltpu.VMEM((n_cam, 10), jnp.float32),
            pltpu.VMEM((B,), jnp.int32),
            pltpu.VMEM((B,), jnp.int32),
            pltpu.VMEM((B, 3), jnp.float32),
            pltpu.VMEM((B,), jnp.float32),
            pltpu.VMEM((B,), jnp.float32),
            pltpu.SemaphoreType.DMA,
        ],
    )
    def reproj(cidx_hbm, pidx_hbm, cam_hbm, pts_hbm, u_hbm, v_hbm,
               cam_v, cidx_v, pidx_v, pts_v, u_v, v_v, sem):
        wid = lax.axis_index("s") * NC + lax.axis_index("c")
        base = jnp.minimum(wid * S, last)
        pltpu.sync_copy(cam_hbm, cam_v)
        iota = lax.iota(jnp.int32, L)

        def col(j):
            return jnp.full((L,), j, jnp.int32)

        def step(si, carry):
            off = pl.multiple_of(base + si * B, 16)
            pltpu.sync_copy(pidx_hbm.at[pl.ds(off, B)], pidx_v)
            cp = pltpu.async_copy(pts_hbm.at[pidx_v], pts_v, sem)
            pltpu.sync_copy(cidx_hbm.at[pl.ds(off, B)], cidx_v)
            cp.wait()

            def inner(k, c):
                rows = k * L + iota
                ci = cidx_v[pl.ds(k * L, L)]
                qw = plsc.load_gather(cam_v, [ci, col(0)])
                qx = plsc.load_gather(cam_v, [ci, col(1)])
                qy = plsc.load_gather(cam_v, [ci, col(2)])
                qz = plsc.load_gather(cam_v, [ci, col(3)])
                tx = plsc.load_gather(cam_v, [ci, col(4)])
                ty = plsc.load_gather(cam_v, [ci, col(5)])
                tz = plsc.load_gather(cam_v, [ci, col(6)])
                fo = plsc.load_gather(cam_v, [ci, col(7)])
                k1 = plsc.load_gather(cam_v, [ci, col(8)])
                k2 = plsc.load_gather(cam_v, [ci, col(9)])
                px = plsc.load_gather(pts_v, [rows, col(0)])
                py = plsc.load_gather(pts_v, [rows, col(1)])
                pz = plsc.load_gather(pts_v, [rows, col(2)])
                s = qw * qw + qx * qx + qy * qy + qz * qz
                inv = 2.0 / s
                t1 = qy * pz - qz * py + qw * px
                t2 = qz * px - qx * pz + qw * py
                t3 = qx * py - qy * px + qw * pz
                c1 = qy * t3 - qz * t2
                c2 = qz * t1 - qx * t3
                c3 = qx * t2 - qy * t1
                x = px + inv * c1 + tx
                y = py + inv * c2 + ty
                z = pz + inv * c3 + tz
                invz = -1.0 / z
                u = x * invz
                v = y * invz
                n = u * u + v * v
                r = 1.0 + k1 * n + k2 * (n * n)
                rf = r * fo
                u_v[pl.ds(k * L, L)] = u * rf
                v_v[pl.ds(k * L, L)] = v * rf
                return c

            lax.fori_loop(0, B // L, inner, 0)
            pltpu.sync_copy(u_v, u_hbm.at[pl.ds(off, B)])
            pltpu.sync_copy(v_v, v_hbm.at[pl.ds(off, B)])
            return carry

        lax.fori_loop(0, nsteps, step, 0)

    return reproj


def kernel(points_2d, camera_indices, point_indices, camera_params, points_3d):
    n_obs = points_2d.shape[0]
    fn = _make_kernel(n_obs, camera_params.shape[0], points_3d.shape[0])
    u, v = fn(camera_indices.astype(jnp.int32),
              point_indices.astype(jnp.int32),
              camera_params.astype(jnp.float32),
              points_3d.astype(jnp.float32))
    return jnp.stack([u, v], axis=-1) - points_2d.astype(jnp.float32)


# speedup vs baseline: 1.2606x; 1.2606x over previous
import functools

import jax
import jax.numpy as jnp
from jax import lax
from jax.experimental import pallas as pl
from jax.experimental.pallas import tpu as pltpu
from jax.experimental.pallas import tpu_sc as plsc

NC, NS, L = 2, 16, 16
NW = NC * NS


def _ceil_to(x, m):
    return (x + m - 1) // m * m


@functools.lru_cache(maxsize=None)
def _make_kernel(n_obs, n_cam, n_pts):
    B = 3136
    nsteps = -(-n_obs // (NW * B))
    C = nsteps * B
    S = _ceil_to(-(-(n_obs - C) // (NW - 1)), 16)
    last = n_obs - C

    mesh = plsc.VectorSubcoreMesh(
        core_axis_name="c", subcore_axis_name="s",
        num_cores=NC, num_subcores=NS)

    vb = lambda n, dt: pltpu.VMEM((n,), dt)

    @functools.partial(
        pl.kernel,
        out_type=(jax.ShapeDtypeStruct((n_obs,), jnp.float32),
                  jax.ShapeDtypeStruct((n_obs,), jnp.float32)),
        mesh=mesh,
        compiler_params=pltpu.CompilerParams(
            needs_layout_passes=False, use_tc_tiling_on_sc=False,
            disable_bounds_checks=True),
        scratch_types=[
            vb(10 * n_cam, jnp.float32),
            [vb(B, jnp.int32)] * 2,
            [vb(B, jnp.int32)] * 2,
            [vb(3 * B, jnp.int32)] * 2,
            [vb(3 * B, jnp.float32)] * 2,
            [vb(B, jnp.float32)] * 2,
            [vb(B, jnp.float32)] * 2,
            pltpu.SemaphoreType.DMA,
            pltpu.SemaphoreType.DMA,
        ],
    )
    def reproj(cidx_hbm, pidx_hbm, cam_hbm, pts_hbm, u_hbm, v_hbm,
               cam_v, cidx_v, pidx_v, i3_v, pts_v, u_v, v_v, gsem, osem):
        wid = lax.axis_index("s") * NC + lax.axis_index("c")
        base = jnp.minimum(wid * S, last)
        pltpu.sync_copy(cam_hbm, cam_v)

        def stage(g, b):
            off = pl.multiple_of(base + g * B, 16)
            pltpu.sync_copy(pidx_hbm.at[pl.ds(off, B)], pidx_v[b])

            def expand(k, c):
                p3 = pidx_v[b][pl.ds(k * L, L)] * 3
                i3_v[b][pl.ds(k * L, L)] = p3
                i3_v[b][pl.ds(B + k * L, L)] = p3 + 1
                i3_v[b][pl.ds(2 * B + k * L, L)] = p3 + 2
                return c

            lax.fori_loop(0, B // L, expand, 0)
            cp = pltpu.async_copy(pts_hbm.at[i3_v[b]], pts_v[b], gsem)
            pltpu.sync_copy(cidx_hbm.at[pl.ds(off, B)], cidx_v[b])
            return cp

        def compute(b):
            def inner(k, c):
                ci = cidx_v[b][pl.ds(k * L, L)]
                cb = ci * 10
                qw = plsc.load_gather(cam_v, [cb])
                qx = plsc.load_gather(cam_v, [cb + 1])
                qy = plsc.load_gather(cam_v, [cb + 2])
                qz = plsc.load_gather(cam_v, [cb + 3])
                tx = plsc.load_gather(cam_v, [cb + 4])
                ty = plsc.load_gather(cam_v, [cb + 5])
                tz = plsc.load_gather(cam_v, [cb + 6])
                fo = plsc.load_gather(cam_v, [cb + 7])
                k1 = plsc.load_gather(cam_v, [cb + 8])
                k2 = plsc.load_gather(cam_v, [cb + 9])
                px = pts_v[b][pl.ds(k * L, L)]
                py = pts_v[b][pl.ds(B + k * L, L)]
                pz = pts_v[b][pl.ds(2 * B + k * L, L)]
                s = qw * qw + qx * qx + qy * qy + qz * qz
                inv = 2.0 / s
                t1 = qy * pz - qz * py + qw * px
                t2 = qz * px - qx * pz + qw * py
                t3 = qx * py - qy * px + qw * pz
                c1 = qy * t3 - qz * t2
                c2 = qz * t1 - qx * t3
                c3 = qx * t2 - qy * t1
                x = px + inv * c1 + tx
                y = py + inv * c2 + ty
                z = pz + inv * c3 + tz
                invz = -1.0 / z
                u = x * invz
                v = y * invz
                n = u * u + v * v
                r = 1.0 + k1 * n + k2 * (n * n)
                rf = r * fo
                u_v[b][pl.ds(k * L, L)] = u * rf
                v_v[b][pl.ds(k * L, L)] = v * rf
                return c

            lax.fori_loop(0, B // L, inner, 0)

        gathers = [None, None]
        outs = [None, None]
        gathers[0] = stage(0, 0)
        for g in range(nsteps):
            b = g % 2
            if g + 1 < nsteps:
                gathers[1 - b] = stage(g + 1, 1 - b)
            gathers[b].wait()
            if outs[b] is not None:
                for c in outs[b]:
                    c.wait()
            compute(b)
            off = pl.multiple_of(base + g * B, 16)
            outs[b] = (
                pltpu.async_copy(u_v[b], u_hbm.at[pl.ds(off, B)], osem),
                pltpu.async_copy(v_v[b], v_hbm.at[pl.ds(off, B)], osem),
            )
        for o in outs:
            if o is not None:
                for c in o:
                    c.wait()

    return reproj


def kernel(points_2d, camera_indices, point_indices, camera_params, points_3d):
    n_obs = points_2d.shape[0]
    fn = _make_kernel(n_obs, camera_params.shape[0], points_3d.shape[0])
    u, v = fn(camera_indices.astype(jnp.int32),
              point_indices.astype(jnp.int32),
              camera_params.astype(jnp.float32).reshape(-1),
              points_3d.astype(jnp.float32).reshape(-1))
    return jnp.stack([u, v], axis=-1) - points_2d.astype(jnp.float32)


# speedup vs baseline: 1.9826x; 1.5727x over previous
import functools

import jax
import jax.numpy as jnp
from jax import lax
from jax.experimental import pallas as pl
from jax.experimental.pallas import tpu as pltpu
from jax.experimental.pallas import tpu_sc as plsc

NC, NS, L = 2, 16, 16
NW = NC * NS


def _ceil_to(x, m):
    return (x + m - 1) // m * m


@functools.lru_cache(maxsize=None)
def _make_kernel(n_obs, n_cam, n_pts):
    B = 3136
    nsteps = -(-n_obs // (NW * B))
    C = nsteps * B
    S = _ceil_to(-(-(n_obs - C) // (NW - 1)), 16)
    last = n_obs - C

    mesh = plsc.VectorSubcoreMesh(
        core_axis_name="c", subcore_axis_name="s",
        num_cores=NC, num_subcores=NS)

    vb = lambda n, dt: pltpu.VMEM((n,), dt)

    @functools.partial(
        pl.kernel,
        out_type=(jax.ShapeDtypeStruct((n_obs,), jnp.float32),
                  jax.ShapeDtypeStruct((n_obs,), jnp.float32)),
        mesh=mesh,
        compiler_params=pltpu.CompilerParams(
            needs_layout_passes=False, use_tc_tiling_on_sc=False,
            disable_bounds_checks=True),
        scratch_types=[
            vb(10 * n_cam, jnp.float32),
            [vb(B, jnp.int32)] * 2,
            [vb(B, jnp.int32)] * 2,
            [vb(3 * B, jnp.int32)] * 2,
            [vb(3 * B, jnp.float32)] * 2,
            [vb(B, jnp.float32)] * 2,
            [vb(B, jnp.float32)] * 2,
            pltpu.SemaphoreType.DMA,
            pltpu.SemaphoreType.DMA,
        ],
    )
    def reproj(cidx_hbm, pidx_hbm, cam_hbm, pts_hbm, u_hbm, v_hbm,
               cam_v, cidx_v, pidx_v, i3_v, pts_v, u_v, v_v, gsem, osem):
        wid = lax.axis_index("s") * NC + lax.axis_index("c")
        base = jnp.minimum(wid * S, last)
        pltpu.sync_copy(cam_hbm, cam_v)

        def stage(g, b):
            off = pl.multiple_of(base + g * B, 16)
            pltpu.sync_copy(pidx_hbm.at[pl.ds(off, B)], pidx_v[b])

            def expand(k, c):
                p = pidx_v[b][pl.ds(k * L, L)]
                i3_v[b][pl.ds(k * L, L)] = p
                i3_v[b][pl.ds(B + k * L, L)] = p + n_pts
                i3_v[b][pl.ds(2 * B + k * L, L)] = p + 2 * n_pts
                return c

            lax.fori_loop(0, B // L, expand, 0)
            cp = pltpu.async_copy(pts_hbm.at[i3_v[b]], pts_v[b], gsem)
            pltpu.sync_copy(cidx_hbm.at[pl.ds(off, B)], cidx_v[b])
            return cp

        def compute(b):
            def inner(k, c):
                ci = cidx_v[b][pl.ds(k * L, L)]
                cb = ci * 10
                qw = plsc.load_gather(cam_v, [cb])
                qx = plsc.load_gather(cam_v, [cb + 1])
                qy = plsc.load_gather(cam_v, [cb + 2])
                qz = plsc.load_gather(cam_v, [cb + 3])
                tx = plsc.load_gather(cam_v, [cb + 4])
                ty = plsc.load_gather(cam_v, [cb + 5])
                tz = plsc.load_gather(cam_v, [cb + 6])
                fo = plsc.load_gather(cam_v, [cb + 7])
                k1 = plsc.load_gather(cam_v, [cb + 8])
                k2 = plsc.load_gather(cam_v, [cb + 9])
                px = pts_v[b][pl.ds(k * L, L)]
                py = pts_v[b][pl.ds(B + k * L, L)]
                pz = pts_v[b][pl.ds(2 * B + k * L, L)]
                s = qw * qw + qx * qx + qy * qy + qz * qz
                inv = 2.0 / s
                t1 = qy * pz - qz * py + qw * px
                t2 = qz * px - qx * pz + qw * py
                t3 = qx * py - qy * px + qw * pz
                c1 = qy * t3 - qz * t2
                c2 = qz * t1 - qx * t3
                c3 = qx * t2 - qy * t1
                x = px + inv * c1 + tx
                y = py + inv * c2 + ty
                z = pz + inv * c3 + tz
                invz = -1.0 / z
                u = x * invz
                v = y * invz
                n = u * u + v * v
                r = 1.0 + k1 * n + k2 * (n * n)
                rf = r * fo
                u_v[b][pl.ds(k * L, L)] = u * rf
                v_v[b][pl.ds(k * L, L)] = v * rf
                return c

            lax.fori_loop(0, B // L, inner, 0)

        gathers = [None, None]
        outs = [None, None]
        gathers[0] = stage(0, 0)
        for g in range(nsteps):
            b = g % 2
            if g + 1 < nsteps:
                gathers[1 - b] = stage(g + 1, 1 - b)
            gathers[b].wait()
            if outs[b] is not None:
                for c in outs[b]:
                    c.wait()
            compute(b)
            off = pl.multiple_of(base + g * B, 16)
            outs[b] = (
                pltpu.async_copy(u_v[b], u_hbm.at[pl.ds(off, B)], osem),
                pltpu.async_copy(v_v[b], v_hbm.at[pl.ds(off, B)], osem),
            )
        for o in outs:
            if o is not None:
                for c in o:
                    c.wait()

    return reproj


def kernel(points_2d, camera_indices, point_indices, camera_params, points_3d):
    n_obs = points_2d.shape[0]
    fn = _make_kernel(n_obs, camera_params.shape[0], points_3d.shape[0])
    u, v = fn(camera_indices.astype(jnp.int32),
              point_indices.astype(jnp.int32),
              camera_params.astype(jnp.float32).reshape(-1),
              points_3d.astype(jnp.float32).T.reshape(-1))
    return jnp.stack([u, v], axis=-1) - points_2d.astype(jnp.float32)
